# baseline (speedup 1.0000x reference)
import jax
import jax.numpy as jnp
from jax import lax
from jax.experimental import pallas as pl
from jax.experimental.pallas import tpu as pltpu
from jax.experimental.pallas import tpu_sc as plsc

N = 10000
E = 320000
D = 128
D_EDGE = 16

NC = 2
NS = 16
EPS = E // NS
CHUNK = 200
NCHUNK = EPS // CHUNK


def _pre_body(s_ref, r_ref, w0s_ref, w0r_ref, p_ref):
    p_ref[0] = jnp.dot(s_ref[...], w0s_ref[...],
                       preferred_element_type=jnp.float32)
    p_ref[1] = jnp.dot(r_ref[...], w0r_ref[...],
                       preferred_element_type=jnp.float32)


def _precompute(sender_features, receiver_features, w0s, w0r):
    blk = 2000
    grid = (N // blk,)
    return pl.pallas_call(
        _pre_body,
        grid=grid,
        in_specs=[
            pl.BlockSpec((blk, D), lambda i: (i, 0)),
            pl.BlockSpec((blk, D), lambda i: (i, 0)),
            pl.BlockSpec((D, D), lambda i: (0, 0)),
            pl.BlockSpec((D, D), lambda i: (0, 0)),
        ],
        out_specs=pl.BlockSpec((NC, blk, D), lambda i: (0, i, 0)),
        out_shape=jax.ShapeDtypeStruct((NC, N, D), jnp.float32),
    )(sender_features, receiver_features, w0s, w0r)


NBUF = 4


def _sc_gather_body(tables_hbm, s_hbm, r_hbm, g_hbm,
                    idx_all, r0, r1, r2, r3,
                    sg0, sg1, sg2, sg3, sw0, sw1, sw2, sw3):
    core = lax.axis_index("c")
    sid = lax.axis_index("s")
    base = sid * EPS
    rows = (r0, r1, r2, r3)
    sem_g = (sg0, sg1, sg2, sg3)
    sem_w = (sw0, sw1, sw2, sw3)

    def run_core(idx_hbm, slot):
        table = tables_hbm.at[slot]
        out = g_hbm.at[slot]
        pltpu.sync_copy(idx_hbm.at[pl.ds(base, EPS)], idx_all)

        def idx_sl(ch):
            return idx_all.at[pl.ds(ch * CHUNK, CHUNK)]

        def start(ch, b):
            pltpu.async_copy(table.at[idx_sl(ch)], rows[b], sem_g[b])

        def wait_g(b):
            pltpu.make_async_copy(table.at[idx_sl(0)], rows[b],
                                  sem_g[b]).wait()

        def wb(ch, b):
            pltpu.async_copy(rows[b],
                             out.at[pl.ds(base + ch * CHUNK, CHUNK)],
                             sem_w[b])

        def wait_w(b):
            pltpu.make_async_copy(rows[b], out.at[pl.ds(base, CHUNK)],
                                  sem_w[b]).wait()

        for b in range(NBUF):
            start(b, b)

        @pl.loop(0, NCHUNK // NBUF - 1)
        def _(i):
            ch = i * NBUF
            for b in range(NBUF):
                wait_g(b)
                wb(ch + b, b)
            for b in range(NBUF):
                wait_w(b)
                start(ch + NBUF + b, b)

        last = NCHUNK - NBUF
        for b in range(NBUF):
            wait_g(b)
            wb(last + b, b)
        for b in range(NBUF):
            wait_w(b)

    @pl.when(core == 0)
    def _():
        run_core(s_hbm, 0)

    @pl.when(core == 1)
    def _():
        run_core(r_hbm, 1)


def _sc_gather(tables, senders, receivers):
    mesh = plsc.VectorSubcoreMesh(core_axis_name="c", subcore_axis_name="s",
                                  num_cores=NC, num_subcores=NS)
    run = pl.kernel(
        _sc_gather_body,
        out_type=jax.ShapeDtypeStruct((NC, E, D), jnp.float32),
        mesh=mesh,
        scratch_types=(
            [pltpu.VMEM((EPS,), jnp.int32)]
            + [pltpu.VMEM((CHUNK, D), jnp.float32) for _ in range(NBUF)]
            + [pltpu.SemaphoreType.DMA for _ in range(2 * NBUF)]
        ),
    )
    return run(tables, senders, receivers)


def _mlp_body(gs_ref, gr_ref, ef_ref, w0e_ref, b0_ref, w1_ref, b1_ref,
              lns_ref, lnb_ref, out_ref):
    z = (gs_ref[0] + gr_ref[0]
         + jnp.dot(ef_ref[...], w0e_ref[...],
                   preferred_element_type=jnp.float32)
         + b0_ref[...])
    h = jnp.maximum(z, 0.0).astype(jnp.bfloat16)
    o = jnp.dot(h, w1_ref[...],
                preferred_element_type=jnp.float32) + b1_ref[...]
    mu = jnp.mean(o, axis=-1, keepdims=True)
    d = o - mu
    var = jnp.mean(d * d, axis=-1, keepdims=True)
    out_ref[...] = d * lax.rsqrt(var + 1e-6) * lns_ref[...] + lnb_ref[...]


def _mlp(g, ef, w0e, b0, w1, b1, lns, lnb):
    blk = 4000
    grid = (E // blk,)
    full = lambda shape: pl.BlockSpec(shape, lambda i: (0, 0))
    return pl.pallas_call(
        _mlp_body,
        grid=grid,
        in_specs=[
            pl.BlockSpec((1, blk, D), lambda i: (0, i, 0)),
            pl.BlockSpec((1, blk, D), lambda i: (1, i, 0)),
            pl.BlockSpec((blk, D_EDGE), lambda i: (i, 0)),
            full((D_EDGE, D)),
            full((1, D)),
            full((D, D)),
            full((1, D)),
            full((1, D)),
            full((1, D)),
        ],
        out_specs=pl.BlockSpec((blk, D), lambda i: (i, 0)),
        out_shape=jax.ShapeDtypeStruct((E, D), jnp.float32),
    )(g, g, ef, w0e, b0, w1, b1, lns, lnb)


def kernel(sender_features, receiver_features, edge_features, senders,
           receivers, W0, b0, W1, b1, ln_scale, ln_bias):
    w0s = W0[:D]
    w0r = W0[D:2 * D]
    w0e = W0[2 * D:]
    senders = senders.astype(jnp.int32)
    receivers = receivers.astype(jnp.int32)
    tables = _precompute(sender_features, receiver_features, w0s, w0r)
    g = _sc_gather(tables, senders, receivers)
    return _mlp(g, edge_features.astype(jnp.bfloat16),
                w0e.astype(jnp.bfloat16), b0.reshape(1, D),
                W1.astype(jnp.bfloat16), b1.reshape(1, D),
                ln_scale.reshape(1, D), ln_bias.reshape(1, D))
